# baseline (speedup 1.0000x reference)
import functools

import jax
import jax.numpy as jnp
from jax import lax
from jax.experimental import pallas as pl
from jax.experimental.pallas import tpu as pltpu
from jax.experimental.pallas import tpu_sc as plsc

N_NODES = 10000
N_EDGES = 320000
D_FEAT = 128

NC = 2
NS = 16
NW = NC * NS

CHUNK_A = 128
N_CHUNKS_A = N_EDGES // CHUNK_A
CPW_A = N_CHUNKS_A // NW
BATCH_A = 26
N_BATCH_A = CPW_A // BATCH_A
EXTRA_BASE_A = NW * CPW_A
N_EXTRA_A = N_CHUNKS_A - EXTRA_BASE_A

CHUNK = 128
N_CHUNKS_TOT = N_EDGES // CHUNK
CPW = N_CHUNKS_TOT // NW
NPAIR = CPW // 2
EXTRA_BASE = NW * CPW
N_EXTRA = N_CHUNKS_TOT - EXTRA_BASE

ROWS_PER_S = 624
ROWS_REM = N_NODES - NS * ROWS_PER_S
REM_BASE = NS * ROWS_PER_S

_MESH = plsc.VectorSubcoreMesh(core_axis_name="c", subcore_axis_name="s")


@functools.partial(
    pl.kernel,
    out_type=jax.ShapeDtypeStruct((NC, 2, N_NODES), jnp.float32),
    mesh=_MESH,
    scratch_types=[
        pltpu.VMEM((CPW_A, 1, CHUNK_A), jnp.int32),
        pltpu.VMEM((CPW_A, 1, CHUNK_A), jnp.int32),
        pltpu.VMEM((CHUNK_A,), jnp.float32),
        pltpu.VMEM((2, N_NODES), jnp.float32),
        pltpu.VMEM_SHARED((N_NODES,), jnp.float32),
        pltpu.VMEM_SHARED((N_NODES,), jnp.float32),
        pltpu.SemaphoreType.DMA,
    ],
)
def _sc_degrees(src_hbm, dst_hbm, ones_hbm, zeros_hbm, out_hbm,
                sidx, didx, ones_v, stage, deg_s, deg_d, sem):
    c = lax.axis_index("c")
    s = lax.axis_index("s")
    wid = s * NC + c

    @pl.when(s == 0)
    def _():
        pltpu.sync_copy(zeros_hbm, deg_s)
        pltpu.sync_copy(zeros_hbm, deg_d)

    pltpu.sync_copy(src_hbm.at[pl.ds(wid * CPW_A, CPW_A)], sidx)
    pltpu.sync_copy(dst_hbm.at[pl.ds(wid * CPW_A, CPW_A)], didx)
    pltpu.sync_copy(ones_hbm, ones_v)
    plsc.subcore_barrier()

    def drain_one(t, carry):
        pltpu.make_async_copy(
            ones_v, deg_s.at[sidx.at[0, 0]], sem).wait()
        return carry

    def batch(b, carry):
        def fire(t, carry2):
            j = b * BATCH_A + t
            pltpu.async_copy(ones_v, deg_s.at[sidx.at[j, 0]], sem, add=True)
            pltpu.async_copy(ones_v, deg_d.at[didx.at[j, 0]], sem, add=True)
            return carry2

        lax.fori_loop(0, BATCH_A, fire, 0)
        lax.fori_loop(0, 2 * BATCH_A, drain_one, 0)
        return carry

    lax.fori_loop(0, N_BATCH_A, batch, 0)

    @pl.when(wid < N_EXTRA_A)
    def _():
        pltpu.sync_copy(src_hbm.at[pl.ds(EXTRA_BASE_A + wid, 1)], sidx.at[pl.ds(0, 1)])
        pltpu.sync_copy(dst_hbm.at[pl.ds(EXTRA_BASE_A + wid, 1)], didx.at[pl.ds(0, 1)])
        pltpu.sync_copy(ones_v, deg_s.at[sidx.at[0, 0]], add=True)
        pltpu.sync_copy(ones_v, deg_d.at[didx.at[0, 0]], add=True)

    plsc.subcore_barrier()

    @pl.when(s == 0)
    def _():
        pltpu.sync_copy(deg_s, stage.at[0])
        pltpu.sync_copy(deg_d, stage.at[1])
        pltpu.sync_copy(stage, out_hbm.at[c])


@functools.partial(
    pl.kernel,
    out_type=jax.ShapeDtypeStruct((NC, N_NODES, D_FEAT), jnp.float32),
    mesh=_MESH,
    scratch_types=[
        pltpu.VMEM((4, 1, CHUNK), jnp.int32),
        pltpu.VMEM((4, 1, CHUNK), jnp.int32),
        pltpu.VMEM((CHUNK, D_FEAT), jnp.float32),
        pltpu.VMEM((CHUNK, D_FEAT), jnp.float32),
        pltpu.VMEM_SHARED((N_NODES, D_FEAT), jnp.float32),
        pltpu.SemaphoreType.DMA,
        pltpu.SemaphoreType.DMA,
        pltpu.SemaphoreType.DMA,
        pltpu.SemaphoreType.DMA,
        pltpu.SemaphoreType.DMA,
    ],
)
def _sc_scatter(feat_hbm, src_hbm, dst_hbm, zrows_hbm, out_hbm,
                sidx, didx, rows_a, rows_b, acc,
                gsem_a, gsem_b, isem_a, isem_b, zsem):
    c = lax.axis_index("c")
    s = lax.axis_index("s")
    wid = s * NC + c
    base = wid * CPW

    pltpu.async_copy(zrows_hbm, acc.at[pl.ds(s * ROWS_PER_S, ROWS_PER_S)], zsem)

    @pl.when(s == 0)
    def _():
        pltpu.sync_copy(zrows_hbm.at[pl.ds(0, ROWS_REM)],
                        acc.at[pl.ds(REM_BASE, ROWS_REM)])

    def istart(j, slot, isem):
        pltpu.async_copy(src_hbm.at[pl.ds(base + j, 1)], sidx.at[pl.ds(slot, 1)], isem)
        pltpu.async_copy(dst_hbm.at[pl.ds(base + j, 1)], didx.at[pl.ds(slot, 1)], isem)

    def iwait(isem):
        pltpu.make_async_copy(src_hbm.at[pl.ds(0, 1)], sidx.at[pl.ds(0, 1)], isem).wait()
        pltpu.make_async_copy(dst_hbm.at[pl.ds(0, 1)], didx.at[pl.ds(0, 1)], isem).wait()

    def gstart(slot, buf, gsem):
        pltpu.async_copy(feat_hbm.at[sidx.at[slot, 0]], buf, gsem)

    def gwait(buf, gsem):
        pltpu.make_async_copy(feat_hbm.at[sidx.at[0, 0]], buf, gsem).wait()

    def scatter(slot, buf):
        pltpu.sync_copy(buf, acc.at[didx.at[slot, 0]], add=True)

    istart(0, 0, isem_a)
    iwait(isem_a)
    gstart(0, rows_a, gsem_a)
    istart(1, 1, isem_b)
    pltpu.make_async_copy(
        zrows_hbm, acc.at[pl.ds(s * ROWS_PER_S, ROWS_PER_S)], zsem).wait()
    plsc.subcore_barrier()

    def body(i, carry):
        j0 = 2 * i
        s1 = (j0 + 1) & 3
        s2 = (j0 + 2) & 3
        s3 = (j0 + 3) & 3
        gwait(rows_a, gsem_a)
        iwait(isem_b)
        gstart(s1, rows_b, gsem_b)
        scatter(j0 & 3, rows_a)

        @pl.when(i < NPAIR - 1)
        def _():
            istart(j0 + 2, s2, isem_a)

        gwait(rows_b, gsem_b)

        @pl.when(i < NPAIR - 1)
        def _():
            iwait(isem_a)
            gstart(s2, rows_a, gsem_a)

        scatter(s1, rows_b)

        @pl.when(i < NPAIR - 1)
        def _():
            istart(j0 + 3, s3, isem_b)

        return carry

    lax.fori_loop(0, NPAIR, body, 0)

    @pl.when(wid < N_EXTRA)
    def _():
        pltpu.async_copy(src_hbm.at[pl.ds(EXTRA_BASE + wid, 1)],
                         sidx.at[pl.ds(0, 1)], isem_a)
        pltpu.async_copy(dst_hbm.at[pl.ds(EXTRA_BASE + wid, 1)],
                         didx.at[pl.ds(0, 1)], isem_a)
        iwait(isem_a)
        gstart(0, rows_a, gsem_a)
        gwait(rows_a, gsem_a)
        scatter(0, rows_a)

    plsc.subcore_barrier()

    pltpu.sync_copy(acc.at[pl.ds(s * ROWS_PER_S, ROWS_PER_S)],
                    out_hbm.at[c, pl.ds(s * ROWS_PER_S, ROWS_PER_S)])

    @pl.when(s == 0)
    def _():
        pltpu.sync_copy(acc.at[pl.ds(REM_BASE, ROWS_REM)],
                        out_hbm.at[c, pl.ds(REM_BASE, ROWS_REM)])


TC_BLK = 1000
TC_GRID = N_NODES // TC_BLK


def _tc_norm_body(x_ref, deg_ref, feat_ref):
    deg = deg_ref[:, 0] + deg_ref[:, 1]
    norm = lax.rsqrt(jnp.maximum(deg, 1.0))
    feat_ref[...] = x_ref[...] * norm[:, None]


def _tc_final_body(p_ref, deg_ref, out_ref):
    deg = deg_ref[:, 0] + deg_ref[:, 1]
    norm = lax.rsqrt(jnp.maximum(deg, 1.0))
    out_ref[...] = (p_ref[0] + p_ref[1]) * norm[:, None]


def kernel(x, edge_index):
    srcA = edge_index[0].reshape(N_CHUNKS_A, 1, CHUNK_A)
    dstA = edge_index[1].reshape(N_CHUNKS_A, 1, CHUNK_A)
    srcC, dstC = srcA, dstA
    ones = jnp.ones((CHUNK_A,), jnp.float32)
    zeros1 = jnp.zeros((N_NODES,), jnp.float32)
    zrows = jnp.zeros((ROWS_PER_S, D_FEAT), jnp.float32)

    degs = _sc_degrees(srcA, dstA, ones, zeros1)
    deg_src = degs[:, 0, :].T
    deg_dst = degs[:, 1, :].T

    feat = pl.pallas_call(
        _tc_norm_body,
        grid=(TC_GRID,),
        in_specs=[
            pl.BlockSpec((TC_BLK, D_FEAT), lambda i: (i, 0)),
            pl.BlockSpec((TC_BLK, NC), lambda i: (i, 0)),
        ],
        out_specs=pl.BlockSpec((TC_BLK, D_FEAT), lambda i: (i, 0)),
        out_shape=jax.ShapeDtypeStruct((N_NODES, D_FEAT), jnp.float32),
    )(x, deg_src)

    parts = _sc_scatter(feat, srcC, dstC, zrows)

    out = pl.pallas_call(
        _tc_final_body,
        grid=(TC_GRID,),
        in_specs=[
            pl.BlockSpec((NC, TC_BLK, D_FEAT), lambda i: (0, i, 0)),
            pl.BlockSpec((TC_BLK, NC), lambda i: (i, 0)),
        ],
        out_specs=pl.BlockSpec((TC_BLK, D_FEAT), lambda i: (i, 0)),
        out_shape=jax.ShapeDtypeStruct((N_NODES, D_FEAT), jnp.float32),
    )(parts, deg_dst)
    return out


# speedup vs baseline: 1.0879x; 1.0879x over previous
import functools

import jax
import jax.numpy as jnp
from jax import lax
from jax.experimental import pallas as pl
from jax.experimental.pallas import tpu as pltpu
from jax.experimental.pallas import tpu_sc as plsc

N_NODES = 10000
N_EDGES = 320000
D_FEAT = 128

NC = 2
NS = 16
NW = NC * NS

CHUNK_A = 128
N_CHUNKS_A = N_EDGES // CHUNK_A
CPW_A = N_CHUNKS_A // NW
BATCH_A = 26
N_BATCH_A = CPW_A // BATCH_A
EXTRA_BASE_A = NW * CPW_A
N_EXTRA_A = N_CHUNKS_A - EXTRA_BASE_A

CHUNK = 128
N_CHUNKS_TOT = N_EDGES // CHUNK
CPW = N_CHUNKS_TOT // NW
NPAIR = CPW // 2
EXTRA_BASE = NW * CPW
N_EXTRA = N_CHUNKS_TOT - EXTRA_BASE

ROWS_PER_S = 624
ROWS_REM = N_NODES - NS * ROWS_PER_S
REM_BASE = NS * ROWS_PER_S

_MESH = plsc.VectorSubcoreMesh(core_axis_name="c", subcore_axis_name="s")


@functools.partial(
    pl.kernel,
    out_type=jax.ShapeDtypeStruct((NC, 2, N_NODES), jnp.float32),
    mesh=_MESH,
    scratch_types=[
        pltpu.VMEM((CPW_A, 1, CHUNK_A), jnp.int32),
        pltpu.VMEM((CPW_A, 1, CHUNK_A), jnp.int32),
        pltpu.VMEM((CHUNK_A,), jnp.float32),
        pltpu.VMEM((2, N_NODES), jnp.float32),
        pltpu.VMEM_SHARED((N_NODES,), jnp.float32),
        pltpu.VMEM_SHARED((N_NODES,), jnp.float32),
        pltpu.SemaphoreType.DMA,
    ],
)
def _sc_degrees(src_hbm, dst_hbm, ones_hbm, zeros_hbm, out_hbm,
                sidx, didx, ones_v, stage, deg_s, deg_d, sem):
    c = lax.axis_index("c")
    s = lax.axis_index("s")
    wid = s * NC + c

    @pl.when(s == 0)
    def _():
        pltpu.sync_copy(zeros_hbm, deg_s)
        pltpu.sync_copy(zeros_hbm, deg_d)

    pltpu.sync_copy(src_hbm.at[pl.ds(wid * CPW_A, CPW_A)], sidx)
    pltpu.sync_copy(dst_hbm.at[pl.ds(wid * CPW_A, CPW_A)], didx)
    pltpu.sync_copy(ones_hbm, ones_v)
    plsc.subcore_barrier()

    def drain_one(t, carry):
        pltpu.make_async_copy(
            ones_v, deg_s.at[sidx.at[0, 0]], sem).wait()
        return carry

    def batch(b, carry):
        def fire(t, carry2):
            j = b * BATCH_A + t
            pltpu.async_copy(ones_v, deg_s.at[sidx.at[j, 0]], sem, add=True)
            pltpu.async_copy(ones_v, deg_d.at[didx.at[j, 0]], sem, add=True)
            return carry2

        lax.fori_loop(0, BATCH_A, fire, 0)
        lax.fori_loop(0, 2 * BATCH_A, drain_one, 0)
        return carry

    lax.fori_loop(0, N_BATCH_A, batch, 0)

    @pl.when(wid < N_EXTRA_A)
    def _():
        pltpu.sync_copy(src_hbm.at[pl.ds(EXTRA_BASE_A + wid, 1)], sidx.at[pl.ds(0, 1)])
        pltpu.sync_copy(dst_hbm.at[pl.ds(EXTRA_BASE_A + wid, 1)], didx.at[pl.ds(0, 1)])
        pltpu.sync_copy(ones_v, deg_s.at[sidx.at[0, 0]], add=True)
        pltpu.sync_copy(ones_v, deg_d.at[didx.at[0, 0]], add=True)

    plsc.subcore_barrier()

    @pl.when(s == 0)
    def _():
        pltpu.sync_copy(deg_s, stage.at[0])
        pltpu.sync_copy(deg_d, stage.at[1])
        pltpu.sync_copy(stage, out_hbm.at[c])


@functools.partial(
    pl.kernel,
    out_type=jax.ShapeDtypeStruct((NC, N_NODES, D_FEAT), jnp.float32),
    mesh=_MESH,
    scratch_types=[
        pltpu.VMEM((4, 1, CHUNK), jnp.int32),
        pltpu.VMEM((4, 1, CHUNK), jnp.int32),
        pltpu.VMEM((CHUNK, D_FEAT), jnp.float32),
        pltpu.VMEM((CHUNK, D_FEAT), jnp.float32),
        pltpu.VMEM_SHARED((N_NODES, D_FEAT), jnp.float32),
        pltpu.SemaphoreType.DMA,
        pltpu.SemaphoreType.DMA,
        pltpu.SemaphoreType.DMA,
        pltpu.SemaphoreType.DMA,
        pltpu.SemaphoreType.DMA,
    ],
)
def _sc_scatter(feat_hbm, src_hbm, dst_hbm, zrows_hbm, out_hbm,
                sidx, didx, rows_a, rows_b, acc,
                gsem_a, gsem_b, isem_a, isem_b, zsem):
    c = lax.axis_index("c")
    s = lax.axis_index("s")
    wid = s * NC + c
    base = wid * CPW

    pltpu.async_copy(zrows_hbm, acc.at[pl.ds(s * ROWS_PER_S, ROWS_PER_S)], zsem)

    @pl.when(s == 0)
    def _():
        pltpu.sync_copy(zrows_hbm.at[pl.ds(0, ROWS_REM)],
                        acc.at[pl.ds(REM_BASE, ROWS_REM)])

    def istart(j, slot, isem):
        pltpu.async_copy(src_hbm.at[pl.ds(base + j, 1)], sidx.at[pl.ds(slot, 1)], isem)
        pltpu.async_copy(dst_hbm.at[pl.ds(base + j, 1)], didx.at[pl.ds(slot, 1)], isem)

    def iwait(isem):
        pltpu.make_async_copy(src_hbm.at[pl.ds(0, 1)], sidx.at[pl.ds(0, 1)], isem).wait()
        pltpu.make_async_copy(dst_hbm.at[pl.ds(0, 1)], didx.at[pl.ds(0, 1)], isem).wait()

    def gstart(slot, buf, gsem):
        pltpu.async_copy(feat_hbm.at[sidx.at[slot, 0]], buf, gsem)

    def gwait(buf, gsem):
        pltpu.make_async_copy(feat_hbm.at[sidx.at[0, 0]], buf, gsem).wait()

    def scatter(slot, buf):
        pltpu.sync_copy(buf, acc.at[didx.at[slot, 0]], add=True)

    istart(0, 0, isem_a)
    iwait(isem_a)
    gstart(0, rows_a, gsem_a)
    istart(1, 1, isem_b)
    pltpu.make_async_copy(
        zrows_hbm, acc.at[pl.ds(s * ROWS_PER_S, ROWS_PER_S)], zsem).wait()
    plsc.subcore_barrier()

    def body(i, carry):
        j0 = 2 * i
        s1 = (j0 + 1) & 3
        s2 = (j0 + 2) & 3
        s3 = (j0 + 3) & 3
        gwait(rows_a, gsem_a)
        iwait(isem_b)
        gstart(s1, rows_b, gsem_b)
        scatter(j0 & 3, rows_a)

        @pl.when(i < NPAIR - 1)
        def _():
            istart(j0 + 2, s2, isem_a)

        gwait(rows_b, gsem_b)

        @pl.when(i < NPAIR - 1)
        def _():
            iwait(isem_a)
            gstart(s2, rows_a, gsem_a)

        scatter(s1, rows_b)

        @pl.when(i < NPAIR - 1)
        def _():
            istart(j0 + 3, s3, isem_b)

        return carry

    lax.fori_loop(0, NPAIR, body, 0)

    @pl.when(wid < N_EXTRA)
    def _():
        pltpu.async_copy(src_hbm.at[pl.ds(EXTRA_BASE + wid, 1)],
                         sidx.at[pl.ds(0, 1)], isem_a)
        pltpu.async_copy(dst_hbm.at[pl.ds(EXTRA_BASE + wid, 1)],
                         didx.at[pl.ds(0, 1)], isem_a)
        iwait(isem_a)
        gstart(0, rows_a, gsem_a)
        gwait(rows_a, gsem_a)
        scatter(0, rows_a)

    plsc.subcore_barrier()

    pltpu.sync_copy(acc.at[pl.ds(s * ROWS_PER_S, ROWS_PER_S)],
                    out_hbm.at[c, pl.ds(s * ROWS_PER_S, ROWS_PER_S)])

    @pl.when(s == 0)
    def _():
        pltpu.sync_copy(acc.at[pl.ds(REM_BASE, ROWS_REM)],
                        out_hbm.at[c, pl.ds(REM_BASE, ROWS_REM)])


TC_BLK = 1000
TC_GRID = N_NODES // TC_BLK


def _tc_norm_body(x_ref, degs_ref, feat_ref):
    deg = degs_ref[0, 0, :] + degs_ref[1, 0, :]
    norm = lax.rsqrt(jnp.maximum(deg, 1.0))
    feat_ref[...] = x_ref[...] * norm[:, None]


def _tc_final_body(p_ref, degs_ref, out_ref):
    deg = degs_ref[0, 1, :] + degs_ref[1, 1, :]
    norm = lax.rsqrt(jnp.maximum(deg, 1.0))
    out_ref[...] = (p_ref[0] + p_ref[1]) * norm[:, None]


def kernel(x, edge_index):
    srcA = edge_index[0].reshape(N_CHUNKS_A, 1, CHUNK_A)
    dstA = edge_index[1].reshape(N_CHUNKS_A, 1, CHUNK_A)
    srcC, dstC = srcA, dstA
    ones = jnp.ones((CHUNK_A,), jnp.float32)
    zeros1 = jnp.zeros((N_NODES,), jnp.float32)
    zrows = jnp.zeros((ROWS_PER_S, D_FEAT), jnp.float32)

    degs = _sc_degrees(srcA, dstA, ones, zeros1)

    feat = pl.pallas_call(
        _tc_norm_body,
        out_shape=jax.ShapeDtypeStruct((N_NODES, D_FEAT), jnp.float32),
    )(x, degs)

    parts = _sc_scatter(feat, srcC, dstC, zrows)

    out = pl.pallas_call(
        _tc_final_body,
        out_shape=jax.ShapeDtypeStruct((N_NODES, D_FEAT), jnp.float32),
    )(parts, degs)
    return out


# speedup vs baseline: 1.0993x; 1.0105x over previous
import functools

import jax
import jax.numpy as jnp
from jax import lax
from jax.experimental import pallas as pl
from jax.experimental.pallas import tpu as pltpu
from jax.experimental.pallas import tpu_sc as plsc

N_NODES = 10000
N_EDGES = 320000
D_FEAT = 128

NC = 2
NS = 16
NW = NC * NS

CHUNK_A = 128
N_CHUNKS_A = N_EDGES // CHUNK_A
CPW_A = N_CHUNKS_A // NW
BATCH_A = 78
N_BATCH_A = CPW_A // BATCH_A
EXTRA_BASE_A = NW * CPW_A
N_EXTRA_A = N_CHUNKS_A - EXTRA_BASE_A

CHUNK = 128
N_CHUNKS_TOT = N_EDGES // CHUNK
CPW = N_CHUNKS_TOT // NW
NPAIR = CPW // 2
EXTRA_BASE = NW * CPW
N_EXTRA = N_CHUNKS_TOT - EXTRA_BASE

ROWS_PER_S = 624
ROWS_REM = N_NODES - NS * ROWS_PER_S
REM_BASE = NS * ROWS_PER_S

_MESH = plsc.VectorSubcoreMesh(core_axis_name="c", subcore_axis_name="s")


@functools.partial(
    pl.kernel,
    out_type=jax.ShapeDtypeStruct((NC, 2, N_NODES), jnp.float32),
    mesh=_MESH,
    scratch_types=[
        pltpu.VMEM((CPW_A, 1, CHUNK_A), jnp.int32),
        pltpu.VMEM((CPW_A, 1, CHUNK_A), jnp.int32),
        pltpu.VMEM((CHUNK_A,), jnp.float32),
        pltpu.VMEM((2, N_NODES), jnp.float32),
        pltpu.VMEM_SHARED((N_NODES,), jnp.float32),
        pltpu.VMEM_SHARED((N_NODES,), jnp.float32),
        pltpu.SemaphoreType.DMA,
    ],
)
def _sc_degrees(src_hbm, dst_hbm, ones_hbm, zeros_hbm, out_hbm,
                sidx, didx, ones_v, stage, deg_s, deg_d, sem):
    c = lax.axis_index("c")
    s = lax.axis_index("s")
    wid = s * NC + c

    @pl.when(s == 0)
    def _():
        pltpu.sync_copy(zeros_hbm, deg_s)

    @pl.when(s == 1)
    def _():
        pltpu.sync_copy(zeros_hbm, deg_d)

    pltpu.async_copy(src_hbm.at[pl.ds(wid * CPW_A, CPW_A)], sidx, sem)
    pltpu.async_copy(dst_hbm.at[pl.ds(wid * CPW_A, CPW_A)], didx, sem)
    pltpu.sync_copy(ones_hbm, ones_v)
    pltpu.make_async_copy(src_hbm.at[pl.ds(0, CPW_A)], sidx, sem).wait()
    pltpu.make_async_copy(dst_hbm.at[pl.ds(0, CPW_A)], didx, sem).wait()
    plsc.subcore_barrier()

    def drain_one(t, carry):
        pltpu.make_async_copy(
            ones_v, deg_s.at[sidx.at[0, 0]], sem).wait()
        return carry

    def batch(b, carry):
        def fire(t, carry2):
            j = b * BATCH_A + t
            pltpu.async_copy(ones_v, deg_s.at[sidx.at[j, 0]], sem, add=True)
            pltpu.async_copy(ones_v, deg_d.at[didx.at[j, 0]], sem, add=True)
            return carry2

        lax.fori_loop(0, BATCH_A, fire, 0)
        lax.fori_loop(0, 2 * BATCH_A, drain_one, 0)
        return carry

    lax.fori_loop(0, N_BATCH_A, batch, 0)

    @pl.when(wid < N_EXTRA_A)
    def _():
        pltpu.sync_copy(src_hbm.at[pl.ds(EXTRA_BASE_A + wid, 1)], sidx.at[pl.ds(0, 1)])
        pltpu.sync_copy(dst_hbm.at[pl.ds(EXTRA_BASE_A + wid, 1)], didx.at[pl.ds(0, 1)])
        pltpu.sync_copy(ones_v, deg_s.at[sidx.at[0, 0]], add=True)
        pltpu.sync_copy(ones_v, deg_d.at[didx.at[0, 0]], add=True)

    plsc.subcore_barrier()

    @pl.when(s == 0)
    def _():
        pltpu.sync_copy(deg_s, stage.at[0])
        pltpu.sync_copy(deg_d, stage.at[1])
        pltpu.sync_copy(stage, out_hbm.at[c])


@functools.partial(
    pl.kernel,
    out_type=jax.ShapeDtypeStruct((NC, N_NODES, D_FEAT), jnp.float32),
    mesh=_MESH,
    scratch_types=[
        pltpu.VMEM((4, 1, CHUNK), jnp.int32),
        pltpu.VMEM((4, 1, CHUNK), jnp.int32),
        pltpu.VMEM((CHUNK, D_FEAT), jnp.float32),
        pltpu.VMEM((CHUNK, D_FEAT), jnp.float32),
        pltpu.VMEM_SHARED((N_NODES, D_FEAT), jnp.float32),
        pltpu.SemaphoreType.DMA,
        pltpu.SemaphoreType.DMA,
        pltpu.SemaphoreType.DMA,
        pltpu.SemaphoreType.DMA,
        pltpu.SemaphoreType.DMA,
    ],
)
def _sc_scatter(feat_hbm, src_hbm, dst_hbm, zrows_hbm, out_hbm,
                sidx, didx, rows_a, rows_b, acc,
                gsem_a, gsem_b, isem_a, isem_b, zsem):
    c = lax.axis_index("c")
    s = lax.axis_index("s")
    wid = s * NC + c
    base = wid * CPW

    pltpu.async_copy(zrows_hbm, acc.at[pl.ds(s * ROWS_PER_S, ROWS_PER_S)], zsem)

    @pl.when(s == 0)
    def _():
        pltpu.sync_copy(zrows_hbm.at[pl.ds(0, ROWS_REM)],
                        acc.at[pl.ds(REM_BASE, ROWS_REM)])

    def istart(j, slot, isem):
        pltpu.async_copy(src_hbm.at[pl.ds(base + j, 1)], sidx.at[pl.ds(slot, 1)], isem)
        pltpu.async_copy(dst_hbm.at[pl.ds(base + j, 1)], didx.at[pl.ds(slot, 1)], isem)

    def iwait(isem):
        pltpu.make_async_copy(src_hbm.at[pl.ds(0, 1)], sidx.at[pl.ds(0, 1)], isem).wait()
        pltpu.make_async_copy(dst_hbm.at[pl.ds(0, 1)], didx.at[pl.ds(0, 1)], isem).wait()

    def gstart(slot, buf, gsem):
        pltpu.async_copy(feat_hbm.at[sidx.at[slot, 0]], buf, gsem)

    def gwait(buf, gsem):
        pltpu.make_async_copy(feat_hbm.at[sidx.at[0, 0]], buf, gsem).wait()

    def scatter(slot, buf):
        pltpu.sync_copy(buf, acc.at[didx.at[slot, 0]], add=True)

    istart(0, 0, isem_a)
    iwait(isem_a)
    gstart(0, rows_a, gsem_a)
    istart(1, 1, isem_b)
    pltpu.make_async_copy(
        zrows_hbm, acc.at[pl.ds(s * ROWS_PER_S, ROWS_PER_S)], zsem).wait()
    plsc.subcore_barrier()

    def body(i, carry):
        j0 = 2 * i
        s1 = (j0 + 1) & 3
        s2 = (j0 + 2) & 3
        s3 = (j0 + 3) & 3
        gwait(rows_a, gsem_a)
        iwait(isem_b)
        gstart(s1, rows_b, gsem_b)
        scatter(j0 & 3, rows_a)

        @pl.when(i < NPAIR - 1)
        def _():
            istart(j0 + 2, s2, isem_a)

        gwait(rows_b, gsem_b)

        @pl.when(i < NPAIR - 1)
        def _():
            iwait(isem_a)
            gstart(s2, rows_a, gsem_a)

        scatter(s1, rows_b)

        @pl.when(i < NPAIR - 1)
        def _():
            istart(j0 + 3, s3, isem_b)

        return carry

    lax.fori_loop(0, NPAIR, body, 0)

    @pl.when(wid < N_EXTRA)
    def _():
        pltpu.async_copy(src_hbm.at[pl.ds(EXTRA_BASE + wid, 1)],
                         sidx.at[pl.ds(0, 1)], isem_a)
        pltpu.async_copy(dst_hbm.at[pl.ds(EXTRA_BASE + wid, 1)],
                         didx.at[pl.ds(0, 1)], isem_a)
        iwait(isem_a)
        gstart(0, rows_a, gsem_a)
        gwait(rows_a, gsem_a)
        scatter(0, rows_a)

    plsc.subcore_barrier()

    pltpu.sync_copy(acc.at[pl.ds(s * ROWS_PER_S, ROWS_PER_S)],
                    out_hbm.at[c, pl.ds(s * ROWS_PER_S, ROWS_PER_S)])

    @pl.when(s == 0)
    def _():
        pltpu.sync_copy(acc.at[pl.ds(REM_BASE, ROWS_REM)],
                        out_hbm.at[c, pl.ds(REM_BASE, ROWS_REM)])


TC_BLK = 1000
TC_GRID = N_NODES // TC_BLK


def _tc_norm_body(x_ref, degs_ref, feat_ref):
    deg = degs_ref[0, 0, :] + degs_ref[1, 0, :]
    norm = lax.rsqrt(jnp.maximum(deg, 1.0))
    feat_ref[...] = x_ref[...] * norm[:, None]


def _tc_final_body(p_ref, degs_ref, out_ref):
    deg = degs_ref[0, 1, :] + degs_ref[1, 1, :]
    norm = lax.rsqrt(jnp.maximum(deg, 1.0))
    out_ref[...] = (p_ref[0] + p_ref[1]) * norm[:, None]


def kernel(x, edge_index):
    srcA = edge_index[0].reshape(N_CHUNKS_A, 1, CHUNK_A)
    dstA = edge_index[1].reshape(N_CHUNKS_A, 1, CHUNK_A)
    srcC, dstC = srcA, dstA
    ones = jnp.ones((CHUNK_A,), jnp.float32)
    zeros1 = jnp.zeros((N_NODES,), jnp.float32)
    zrows = jnp.zeros((ROWS_PER_S, D_FEAT), jnp.float32)

    degs = _sc_degrees(srcA, dstA, ones, zeros1)

    feat = pl.pallas_call(
        _tc_norm_body,
        out_shape=jax.ShapeDtypeStruct((N_NODES, D_FEAT), jnp.float32),
    )(x, degs)

    parts = _sc_scatter(feat, srcC, dstC, zrows)

    out = pl.pallas_call(
        _tc_final_body,
        out_shape=jax.ShapeDtypeStruct((N_NODES, D_FEAT), jnp.float32),
    )(parts, degs)
    return out
